# baseline (speedup 1.0000x reference)
import functools

import jax
import jax.numpy as jnp
import numpy as np
from jax import lax
from jax.experimental import pallas as pl
from jax.experimental.pallas import tpu as pltpu
from jax.experimental.pallas import tpu_sc as plsc

OUT_H = 56
OUT_W = 56
OUT_HW = OUT_H * OUT_W
NUM_WORKERS = 32
NUM_CORES = 2
LANES = 16
G2 = OUT_HW // LANES

UPL = ((192, 28, 96, 1568, 2), (384, 14, 288, 784, 4), (768, 7, 672, 400, 8))
P1_OFF = (0, 1568, 2352)
P1_TOT = 2752
CK = 8
PAD = 8
VMID_W = PAD + 1568 + 8
VOUT_W = OUT_HW + LANES
SI_OFF = (0, 784)
SI_TOT = 784 + 400
FX_OFF = (0, 224)
FX_TOT = 224 + 448


def _lin_base(n_in: int, n_out: int):
    src = (np.arange(n_out, dtype=np.float64) + 0.5) * (n_in / n_out) - 0.5
    i0 = np.clip(np.floor(src).astype(np.int64), 0, n_in - 2)
    w1 = np.clip(src - i0, 0.0, 1.0)
    return i0, w1


@functools.lru_cache(maxsize=None)
def _tables_r3():
    t1i = np.zeros((P1_TOT,), np.int32)
    t1w = np.zeros((P1_TOT,), np.float32)
    t2i = np.zeros((OUT_HW,), np.int32)
    t2w = np.zeros((OUT_HW,), np.float32)
    tsi = np.full((SI_TOT,), OUT_HW, np.int32)
    tfq = np.zeros((FX_TOT,), np.int32)
    tfw = np.zeros((FX_TOT,), np.float32)
    tfd = np.zeros((FX_TOT,), np.int32)
    for li, (_, n, _, p1p, s) in enumerate(UPL):
        i0, w1 = _lin_base(n, OUT_H)
        p = np.arange(56 * n)
        ho, w = p // n, p % n
        t1i[P1_OFF[li]:P1_OFF[li] + 56 * n] = (i0[ho] * n + w).astype(np.int32)
        t1w[P1_OFF[li]:P1_OFF[li] + 56 * n] = w1[ho].astype(np.float32)
        if li == 0:
            q = np.arange(OUT_HW)
            qho, qwo = q // OUT_W, q % OUT_W
            t2i[:] = (PAD + qho * n + i0[qwo]).astype(np.int32)
            t2w[:] = w1[qwo].astype(np.float32)
        else:
            fi = li - 1
            m = np.arange(56 * n)
            tsi[SI_OFF[fi]:SI_OFF[fi] + 56 * n] = (
                (m // n) * OUT_W + (m % n) * s).astype(np.int32)
            wos = np.concatenate([np.arange(s // 2),
                                  np.arange(OUT_W - s // 2, OUT_W)])
            hos = np.repeat(np.arange(OUT_H), wos.size)
            wos = np.tile(wos, OUT_H)
            sl = slice(FX_OFF[fi], FX_OFF[fi] + 56 * s)
            tfq[sl] = (PAD + hos * n + i0[wos]).astype(np.int32)
            tfw[sl] = w1[wos].astype(np.float32)
            tfd[sl] = (hos * OUT_W + wos).astype(np.int32)
    return tuple(jnp.asarray(a) for a in
                 (t1i, t1w, t2i, t2w, tsi, tfq, tfw, tfd))


def _splat_i32(v):
    return jnp.full((LANES,), v, jnp.int32)


def _sc_body(x0h, x1h, x2h, x3h, t1ih, t1wh, t2ih, t2wh, tsih, tfqh, tfwh,
             tfdh, outh,
             t1i_v, t1w_v, t2i_v, t2w_v, tsi_v, tfq_v, tfw_v, tfd_v,
             vin1a, vin1b, vin2a, vin2b, vin3a, vin3b, vmid, vout0, vout1,
             sem_l0, sem_i0, sem_i1, sem_o0, sem_o1):
    wid = lax.axis_index("s") * NUM_CORES + lax.axis_index("c")
    ivec = lax.iota(jnp.int32, 16)

    c0 = wid * 3

    def l0_issue(b, carry):
        pltpu.async_copy(x0h.at[b, pl.ds(c0, 3)], outh.at[b, pl.ds(c0, 3)],
                         sem_l0)
        return carry

    lax.fori_loop(0, 8, l0_issue, 0)

    pltpu.sync_copy(t1ih, t1i_v)
    pltpu.sync_copy(t1wh, t1w_v)
    pltpu.sync_copy(t2ih, t2i_v)
    pltpu.sync_copy(t2wh, t2w_v)
    pltpu.sync_copy(tsih, tsi_v)
    pltpu.sync_copy(tfqh, tfq_v)
    pltpu.sync_copy(tfwh, tfw_v)
    pltpu.sync_copy(tfdh, tfd_v)

    for li, (xh, vbufs, (C, n, coff, p1p, s)) in enumerate(
            zip((x1h, x2h, x3h),
                ((vin1a, vin1b), (vin2a, vin2b), (vin3a, vin3b)), UPL)):
        vin0, vin1 = vbufs
        tpb = C // CK
        ntasks = 8 * tpb // NUM_WORKERS
        g1 = p1p // LANES
        toff = P1_OFF[li]

        def task_bc(t):
            u = wid * ntasks + t
            return u // tpb, (u % tpb) * CK

        def issue_in(t, vinb, sem):
            b, cs = task_bc(t)
            pltpu.async_copy(xh.at[b, pl.ds(cs, CK)], vinb, sem)

        def wait_in(t, vinb, sem):
            b, cs = task_bc(t)
            pltpu.make_async_copy(xh.at[b, pl.ds(cs, CK)], vinb, sem).wait()

        def compute(vinb, voutb):
            def p1_body(g, carry):
                o = g * LANES
                i0 = t1i_v[pl.ds(toff + o, LANES)]
                w1 = t1w_v[pl.ds(toff + o, LANES)]
                i1 = i0 + n
                w0 = 1.0 - w1
                for c in range(CK):
                    ci = _splat_i32(c)
                    m = (plsc.load_gather(vinb, [ci, i0]) * w0
                         + plsc.load_gather(vinb, [ci, i1]) * w1)
                    vmid[c, pl.ds(PAD + o, LANES)] = m
                return carry

            lax.fori_loop(0, g1, p1_body, 0)

            if li == 0:
                def p2_body(g, carry):
                    o = g * LANES
                    q0 = t2i_v[pl.ds(o, LANES)]
                    w1 = t2w_v[pl.ds(o, LANES)]
                    q1 = q0 + 1
                    w0 = 1.0 - w1
                    for c in range(CK):
                        ci = _splat_i32(c)
                        v = (plsc.load_gather(vmid, [ci, q0]) * w0
                             + plsc.load_gather(vmid, [ci, q1]) * w1)
                        voutb[c, pl.ds(o, LANES)] = v
                    return carry

                lax.fori_loop(0, G2, p2_body, 0)
                return

            fi = li - 1
            soff = SI_OFF[fi]
            half = s // 2

            def sw_body(g, carry):
                o = g * LANES
                sidx = tsi_v[pl.ds(soff + o, LANES)]
                ia = ivec + (PAD - 1 + o)
                ib = ivec + (PAD + o)
                ic = ivec + (PAD + 1 + o)
                for c in range(CK):
                    ci = _splat_i32(c)
                    va = plsc.load_gather(vmid, [ci, ia])
                    vb = plsc.load_gather(vmid, [ci, ib])
                    vc = plsc.load_gather(vmid, [ci, ic])
                    d1 = vb - va
                    d2 = vc - vb
                    for p in range(s):
                        if p < half:
                            v = va + ((2 * p + 1 + s) / (2 * s)) * d1
                        else:
                            v = vb + ((2 * p + 1 - s) / (2 * s)) * d2
                        plsc.store_scatter(voutb, [ci, sidx + p], v)
                return carry

            lax.fori_loop(0, g1, sw_body, 0)

            fxoff = FX_OFF[fi]

            def fx_body(g, carry):
                o = g * LANES
                q0 = tfq_v[pl.ds(fxoff + o, LANES)]
                w1 = tfw_v[pl.ds(fxoff + o, LANES)]
                dst = tfd_v[pl.ds(fxoff + o, LANES)]
                q1 = q0 + 1
                w0 = 1.0 - w1
                for c in range(CK):
                    ci = _splat_i32(c)
                    v = (plsc.load_gather(vmid, [ci, q0]) * w0
                         + plsc.load_gather(vmid, [ci, q1]) * w1)
                    plsc.store_scatter(voutb, [ci, dst], v)
                return carry

            lax.fori_loop(0, 56 * s // LANES, fx_body, 0)

        def issue_out(t, voutb, sem):
            b, cs = task_bc(t)
            pltpu.async_copy(voutb.at[:, pl.ds(0, OUT_HW)],
                             outh.at[b, pl.ds(coff + cs, CK)], sem)

        def drain_out(voutb, sem):
            pltpu.make_async_copy(outh.at[0, pl.ds(0, CK)],
                                  voutb.at[:, pl.ds(0, OUT_HW)], sem).wait()

        issue_in(0, vin0, sem_i0)

        def pair_body(t2, carry):
            for par, vinb, voutb, sem_i, sem_i_nxt, vin_nxt, sem_o in (
                    (0, vin0, vout0, sem_i0, sem_i1, vin1, sem_o0),
                    (1, vin1, vout1, sem_i1, sem_i0, vin0, sem_o1)):
                t = t2 * 2 + par
                wait_in(t, vinb, sem_i)

                @pl.when(t + 1 < ntasks)
                def _():
                    issue_in(t + 1, vin_nxt, sem_i_nxt)

                compute(vinb, voutb)

                @pl.when(t2 > 0)
                def _():
                    drain_out(voutb, sem_o)

                issue_out(t, voutb, sem_o)
            return carry

        lax.fori_loop(0, ntasks // 2, pair_body, 0)
        drain_out(vout0, sem_o0)
        drain_out(vout1, sem_o1)

    def l0_drain(b, carry):
        pltpu.make_async_copy(x0h.at[b, pl.ds(c0, 3)],
                              outh.at[b, pl.ds(c0, 3)], sem_l0).wait()
        return carry

    lax.fori_loop(0, 8, l0_drain, 0)


@jax.jit
def _hypercolumns_sc(x0f, x1f, x2f, x3f, t1i, t1w, t2i, t2w, tsi, tfq, tfw,
                     tfd):
    mesh = plsc.VectorSubcoreMesh(core_axis_name="c", subcore_axis_name="s")
    return pl.kernel(
        _sc_body,
        out_type=jax.ShapeDtypeStruct((8, 1440, OUT_HW), jnp.float32),
        mesh=mesh,
        compiler_params=pltpu.CompilerParams(use_tc_tiling_on_sc=False,
                                             needs_layout_passes=False),
        scratch_types=[
            pltpu.VMEM((P1_TOT,), jnp.int32),
            pltpu.VMEM((P1_TOT,), jnp.float32),
            pltpu.VMEM((OUT_HW,), jnp.int32),
            pltpu.VMEM((OUT_HW,), jnp.float32),
            pltpu.VMEM((SI_TOT,), jnp.int32),
            pltpu.VMEM((FX_TOT,), jnp.int32),
            pltpu.VMEM((FX_TOT,), jnp.float32),
            pltpu.VMEM((FX_TOT,), jnp.int32),
            pltpu.VMEM((CK, 28 * 28), jnp.float32),
            pltpu.VMEM((CK, 28 * 28), jnp.float32),
            pltpu.VMEM((CK, 14 * 14), jnp.float32),
            pltpu.VMEM((CK, 14 * 14), jnp.float32),
            pltpu.VMEM((CK, 7 * 7), jnp.float32),
            pltpu.VMEM((CK, 7 * 7), jnp.float32),
            pltpu.VMEM((CK, VMID_W), jnp.float32),
            pltpu.VMEM((CK, VOUT_W), jnp.float32),
            pltpu.VMEM((CK, VOUT_W), jnp.float32),
            pltpu.SemaphoreType.DMA,
            pltpu.SemaphoreType.DMA,
            pltpu.SemaphoreType.DMA,
            pltpu.SemaphoreType.DMA,
            pltpu.SemaphoreType.DMA,
        ],
    )(x0f, x1f, x2f, x3f, t1i, t1w, t2i, t2w, tsi, tfq, tfw, tfd)


def kernel(x0, x1, x2, x3):
    tabs = _tables_r3()
    x0f = x0.reshape(8, 96, 56 * 56)
    x1f = x1.reshape(8, 192, 28 * 28)
    x2f = x2.reshape(8, 384, 14 * 14)
    x3f = x3.reshape(8, 768, 7 * 7)
    out = _hypercolumns_sc(x0f, x1f, x2f, x3f, *tabs)
    return out.reshape(8, 1440, OUT_H, OUT_W)


# speedup vs baseline: 1.3348x; 1.3348x over previous
import functools

import jax
import jax.numpy as jnp
import numpy as np
from jax import lax
from jax.experimental import pallas as pl
from jax.experimental.pallas import tpu as pltpu
from jax.experimental.pallas import tpu_sc as plsc

OUT_H = 56
OUT_W = 56
NUM_WORKERS = 32
NUM_CORES = 2
LANES = 16

UPL = ((192, 28, 32, 1568), (384, 14, 16, 784), (768, 7, 8, 400))
P1_OFF = (0, 1568, 2352)
P1_TOT = 2752
CK = 8
MID_ROWS = 21504
CBLK = 32
NCB = 1440 // CBLK


def _lin_base(n_in: int, n_out: int):
    src = (np.arange(n_out, dtype=np.float64) + 0.5) * (n_in / n_out) - 0.5
    i0 = np.clip(np.floor(src).astype(np.int64), 0, n_in - 2)
    w1 = np.clip(src - i0, 0.0, 1.0)
    return i0, w1


@functools.lru_cache(maxsize=None)
def _tables_r4():
    t1i = np.zeros((P1_TOT,), np.int32)
    t1w = np.zeros((P1_TOT,), np.float32)
    tds = np.zeros((P1_TOT,), np.int32)
    wms = []
    for li, (_, n, K, p1p) in enumerate(UPL):
        i0, w1 = _lin_base(n, OUT_H)
        p = np.arange(56 * n)
        ho, w = p // n, p % n
        sl = slice(P1_OFF[li], P1_OFF[li] + 56 * n)
        t1i[sl] = (i0[ho] * n + w).astype(np.int32)
        t1w[sl] = w1[ho].astype(np.float32)
        tds[sl] = (ho * K + w).astype(np.int32)
        pad = slice(P1_OFF[li] + 56 * n, P1_OFF[li] + p1p)
        tds[pad] = K - 1
        G = 128 // K
        w1d = np.zeros((K, OUT_W), np.float32)
        for wo in range(OUT_W):
            w1d[i0[wo], wo] += 1.0 - w1[wo]
            w1d[i0[wo] + 1, wo] += w1[wo]
        wm = np.zeros((G, 128, OUT_W), np.float32)
        for j in range(G):
            wm[j, j * K:j * K + K, :] = w1d
        wms.append(jnp.asarray(wm))
    return jnp.asarray(t1i), jnp.asarray(t1w), jnp.asarray(tds), wms


def _splat_i32(v):
    return jnp.full((LANES,), v, jnp.int32)


def _sc_body(x1h, x2h, x3h, t1ih, t1wh, tdsh, m1h, m2h, m3h,
             t1i_v, t1w_v, tds_v,
             vin1a, vin1b, vin2a, vin2b, vin3a, vin3b,
             vm1a, vm1b, vm2a, vm2b, vm3a, vm3b,
             sem_i0, sem_i1, sem_o0, sem_o1):
    wid = lax.axis_index("s") * NUM_CORES + lax.axis_index("c")

    pltpu.sync_copy(t1ih, t1i_v)
    pltpu.sync_copy(t1wh, t1w_v)
    pltpu.sync_copy(tdsh, tds_v)

    for vm, words in ((vm1a, CK * 56 * 32), (vm1b, CK * 56 * 32),
                      (vm2a, CK * 56 * 16), (vm2b, CK * 56 * 16),
                      (vm3a, CK * 56 * 8), (vm3b, CK * 56 * 8)):
        zero = jnp.zeros((LANES,), jnp.float32)

        def zbody(i, carry, vm=vm):
            vm[pl.ds(i * LANES, LANES)] = zero
            return carry

        lax.fori_loop(0, (words + LANES) // LANES, zbody, 0)

    for li, (xh, mh, vbufs, mbufs, (C, n, K, p1p)) in enumerate(
            zip((x1h, x2h, x3h), (m1h, m2h, m3h),
                ((vin1a, vin1b), (vin2a, vin2b), (vin3a, vin3b)),
                ((vm1a, vm1b), (vm2a, vm2b), (vm3a, vm3b)), UPL)):
        vin0, vin1 = vbufs
        vm0, vm1 = mbufs
        n2 = n * n
        tpb = C // CK
        ntasks = 8 * tpb // NUM_WORKERS
        g1 = p1p // LANES
        toff = P1_OFF[li]
        tlen = CK * 56 * K
        rowk = 56 * K

        def task_off(t):
            u = wid * ntasks + t
            b = u // tpb
            cs = (u % tpb) * CK
            return (b * C + cs) * n2, (b * C + cs) * 56 * K

        def issue_in(t, vinb, sem):
            ioff, _ = task_off(t)
            pltpu.async_copy(xh.at[pl.ds(ioff, CK * n2)], vinb, sem)

        def wait_in(t, vinb, sem):
            ioff, _ = task_off(t)
            pltpu.make_async_copy(xh.at[pl.ds(ioff, CK * n2)], vinb,
                                  sem).wait()

        def compute(vinb, vmb):
            def p1_body(g, carry):
                o = g * LANES
                i0 = t1i_v[pl.ds(toff + o, LANES)]
                w1 = t1w_v[pl.ds(toff + o, LANES)]
                dst = tds_v[pl.ds(toff + o, LANES)]
                w0 = 1.0 - w1
                for c in range(CK):
                    iv0 = i0 + c * n2
                    m = (plsc.load_gather(vinb, [iv0]) * w0
                         + plsc.load_gather(vinb, [iv0 + n]) * w1)
                    plsc.store_scatter(vmb, [dst + c * rowk], m)
                return carry

            lax.fori_loop(0, g1, p1_body, 0)

        def issue_out(t, vmb, sem):
            _, ooff = task_off(t)
            pltpu.async_copy(vmb.at[pl.ds(0, tlen)], mh.at[pl.ds(ooff, tlen)],
                             sem)

        def drain_out(vmb, sem):
            pltpu.make_async_copy(mh.at[pl.ds(0, tlen)],
                                  vmb.at[pl.ds(0, tlen)], sem).wait()

        issue_in(0, vin0, sem_i0)

        def pair_body(t2, carry):
            for par, vinb, vmb, sem_i, sem_i_nxt, vin_nxt, sem_o in (
                    (0, vin0, vm0, sem_i0, sem_i1, vin1, sem_o0),
                    (1, vin1, vm1, sem_i1, sem_i0, vin0, sem_o1)):
                t = t2 * 2 + par
                wait_in(t, vinb, sem_i)

                @pl.when(t + 1 < ntasks)
                def _():
                    issue_in(t + 1, vin_nxt, sem_i_nxt)

                @pl.when(t2 > 0)
                def _():
                    drain_out(vmb, sem_o)

                compute(vinb, vmb)
                issue_out(t, vmb, sem_o)
            return carry

        lax.fori_loop(0, ntasks // 2, pair_body, 0)
        drain_out(vm0, sem_o0)
        drain_out(vm1, sem_o1)


def _tc_body(w1_ref, w2_ref, w3_ref, x0_ref, m1_ref, m2_ref, m3_ref, o_ref):
    cb = pl.program_id(1)

    @pl.when(cb < 3)
    def _():
        o_ref[...] = x0_ref[...]

    def expand(m_ref, w_ref, K):
        G = 128 // K
        rows = CBLK * 56 // G
        parts = [
            jax.lax.dot_general(m_ref[...], w_ref[j],
                                (((1,), (0,)), ((), ())),
                                preferred_element_type=jnp.float32)
            for j in range(G)
        ]
        st = jnp.stack(parts, axis=1)
        return st.reshape(1, CBLK, OUT_H, OUT_W)

    @pl.when(jnp.logical_and(cb >= 3, cb < 9))
    def _():
        o_ref[...] = expand(m1_ref, w1_ref, 32)

    @pl.when(jnp.logical_and(cb >= 9, cb < 21))
    def _():
        o_ref[...] = expand(m2_ref, w2_ref, 16)

    @pl.when(cb >= 21)
    def _():
        o_ref[...] = expand(m3_ref, w3_ref, 8)


@jax.jit
def _hypercolumns(x0, x1f, x2f, x3f, t1i, t1w, tds, w1, w2, w3):
    mesh = plsc.VectorSubcoreMesh(core_axis_name="c", subcore_axis_name="s")
    mids = pl.kernel(
        _sc_body,
        out_type=(jax.ShapeDtypeStruct((MID_ROWS * 128,), jnp.float32),
                  jax.ShapeDtypeStruct((MID_ROWS * 128,), jnp.float32),
                  jax.ShapeDtypeStruct((MID_ROWS * 128,), jnp.float32)),
        mesh=mesh,
        compiler_params=pltpu.CompilerParams(use_tc_tiling_on_sc=False,
                                             needs_layout_passes=False),
        scratch_types=[
            pltpu.VMEM((P1_TOT,), jnp.int32),
            pltpu.VMEM((P1_TOT,), jnp.float32),
            pltpu.VMEM((P1_TOT,), jnp.int32),
            pltpu.VMEM((CK * 28 * 28,), jnp.float32),
            pltpu.VMEM((CK * 28 * 28,), jnp.float32),
            pltpu.VMEM((CK * 14 * 14,), jnp.float32),
            pltpu.VMEM((CK * 14 * 14,), jnp.float32),
            pltpu.VMEM((CK * 7 * 7,), jnp.float32),
            pltpu.VMEM((CK * 7 * 7,), jnp.float32),
            pltpu.VMEM((CK * 56 * 32 + LANES,), jnp.float32),
            pltpu.VMEM((CK * 56 * 32 + LANES,), jnp.float32),
            pltpu.VMEM((CK * 56 * 16 + LANES,), jnp.float32),
            pltpu.VMEM((CK * 56 * 16 + LANES,), jnp.float32),
            pltpu.VMEM((CK * 56 * 8 + LANES,), jnp.float32),
            pltpu.VMEM((CK * 56 * 8 + LANES,), jnp.float32),
            pltpu.SemaphoreType.DMA,
            pltpu.SemaphoreType.DMA,
            pltpu.SemaphoreType.DMA,
            pltpu.SemaphoreType.DMA,
        ],
    )(x1f, x2f, x3f, t1i, t1w, tds)
    m1 = mids[0].reshape(MID_ROWS, 128)
    m2 = mids[1].reshape(MID_ROWS, 128)
    m3 = mids[2].reshape(MID_ROWS, 128)
    out = pl.pallas_call(
        _tc_body,
        out_shape=jax.ShapeDtypeStruct((8, 1440, OUT_H, OUT_W), jnp.float32),
        grid=(8, NCB),
        in_specs=[
            pl.BlockSpec((4, 128, OUT_W), lambda b, cb: (0, 0, 0)),
            pl.BlockSpec((8, 128, OUT_W), lambda b, cb: (0, 0, 0)),
            pl.BlockSpec((16, 128, OUT_W), lambda b, cb: (0, 0, 0)),
            pl.BlockSpec((1, CBLK, OUT_H, OUT_W),
                         lambda b, cb: (b, jnp.minimum(cb, 2), 0, 0)),
            pl.BlockSpec((CBLK * 56 // 4, 128),
                         lambda b, cb: (b * 6 + jnp.clip(cb - 3, 0, 5), 0)),
            pl.BlockSpec((CBLK * 56 // 8, 128),
                         lambda b, cb: (b * 12 + jnp.clip(cb - 9, 0, 11), 0)),
            pl.BlockSpec((CBLK * 56 // 16, 128),
                         lambda b, cb: (b * 24 + jnp.clip(cb - 21, 0, 23), 0)),
        ],
        out_specs=pl.BlockSpec((1, CBLK, OUT_H, OUT_W),
                               lambda b, cb: (b, cb, 0, 0)),
    )(w1, w2, w3, x0, m1, m2, m3)
    return out


def kernel(x0, x1, x2, x3):
    t1i, t1w, tds, (w1, w2, w3) = _tables_r4()
    x1f = x1.reshape(-1)
    x2f = x2.reshape(-1)
    x3f = x3.reshape(-1)
    return _hypercolumns(x0, x1f, x2f, x3f, t1i, t1w, tds, w1, w2, w3)


# speedup vs baseline: 1.5200x; 1.1388x over previous
import functools

import jax
import jax.numpy as jnp
import numpy as np
from jax import lax
from jax.experimental import pallas as pl
from jax.experimental.pallas import tpu as pltpu
from jax.experimental.pallas import tpu_sc as plsc

OUT_H = 56
OUT_W = 56
NUM_WORKERS = 32
NUM_CORES = 2
LANES = 16
KP = 32
ROWK = OUT_H * KP
G = 128 // KP

UPL = ((192, 28, 1568), (384, 14, 784), (768, 7, 400))
P1_OFF = (0, 1568, 2352)
P1_TOT = 2752
CK = 8
MROWS = (21504, 43008, 86016)
CBLK = 32
NCB = 1440 // CBLK


def _lin_base(n_in: int, n_out: int):
    src = (np.arange(n_out, dtype=np.float64) + 0.5) * (n_in / n_out) - 0.5
    i0 = np.clip(np.floor(src).astype(np.int64), 0, n_in - 2)
    w1 = np.clip(src - i0, 0.0, 1.0)
    return i0, w1


@functools.lru_cache(maxsize=None)
def _tables_r5():
    t1i = np.zeros((P1_TOT,), np.int32)
    t1w = np.zeros((P1_TOT,), np.float32)
    tdr = np.zeros((P1_TOT,), np.int32)
    tdk = np.zeros((P1_TOT,), np.int32)
    wms = []
    for li, (_, n, p1p) in enumerate(UPL):
        i0, w1 = _lin_base(n, OUT_H)
        p = np.arange(56 * n)
        ho, w = p // n, p % n
        sl = slice(P1_OFF[li], P1_OFF[li] + 56 * n)
        t1i[sl] = (i0[ho] * n + w).astype(np.int32)
        t1w[sl] = w1[ho].astype(np.float32)
        tdr[sl] = ho.astype(np.int32)
        tdk[sl] = w.astype(np.int32)
        pad = slice(P1_OFF[li] + 56 * n, P1_OFF[li] + p1p)
        tdr[pad] = 0
        tdk[pad] = KP - 1
        w1d = np.zeros((KP, OUT_W), np.float32)
        for wo in range(OUT_W):
            w1d[i0[wo], wo] += 1.0 - w1[wo]
            w1d[i0[wo] + 1, wo] += w1[wo]
        wm = np.zeros((G, 128, OUT_W), np.float32)
        for j in range(G):
            wm[j, j * KP:(j + 1) * KP, :] = w1d
        wms.append(jnp.asarray(wm))
    return (jnp.asarray(t1i), jnp.asarray(t1w), jnp.asarray(tdr),
            jnp.asarray(tdk), wms)


def _sc_body(x1h, x2h, x3h, t1ih, t1wh, tdrh, tdkh, m1h, m2h, m3h,
             t1i_v, t1w_v, tdr_v, tdk_v,
             vin1a, vin1b, vin2a, vin2b, vin3a, vin3b,
             vma, vmb_,
             sem_i0, sem_i1, sem_o0, sem_o1):
    wid = lax.axis_index("s") * NUM_CORES + lax.axis_index("c")

    pltpu.sync_copy(t1ih, t1i_v)
    pltpu.sync_copy(t1wh, t1w_v)
    pltpu.sync_copy(tdrh, tdr_v)
    pltpu.sync_copy(tdkh, tdk_v)

    zero = jnp.zeros((LANES,), jnp.float32)
    for vm in (vma, vmb_):
        def zbody(i, carry, vm=vm):
            r = i // 2
            vm[r, pl.ds((i % 2) * LANES, LANES)] = zero
            return carry

        lax.fori_loop(0, 448 * 2, zbody, 0)

    for li, (xh, mh, vbufs, (C, n, p1p)) in enumerate(
            zip((x1h, x2h, x3h), (m1h, m2h, m3h),
                ((vin1a, vin1b), (vin2a, vin2b), (vin3a, vin3b)), UPL)):
        vin0, vin1 = vbufs
        n2 = n * n
        tpb = C // CK
        ntasks = 8 * tpb // NUM_WORKERS
        g1 = p1p // LANES
        toff = P1_OFF[li]

        def task_pos(t):
            u = wid * ntasks + t
            b = u // tpb
            cs = (u % tpb) * CK
            gc = b * C + cs
            return gc * n2, (gc // 32) * 448, ((gc % 32) // 8) * KP

        def issue_in(t, vinb, sem):
            ioff, _, _ = task_pos(t)
            pltpu.async_copy(xh.at[pl.ds(ioff, CK * n2)], vinb, sem)

        def wait_in(t, vinb, sem):
            ioff, _, _ = task_pos(t)
            pltpu.make_async_copy(xh.at[pl.ds(ioff, CK * n2)], vinb,
                                  sem).wait()

        def compute(vinb, vm):
            def p1_body(g, carry):
                o = g * LANES
                i0 = t1i_v[pl.ds(toff + o, LANES)]
                w1 = t1w_v[pl.ds(toff + o, LANES)]
                dr = tdr_v[pl.ds(toff + o, LANES)]
                dk = tdk_v[pl.ds(toff + o, LANES)]
                w0 = 1.0 - w1
                for c in range(CK):
                    iv0 = i0 + c * n2
                    m = (plsc.load_gather(vinb, [iv0]) * w0
                         + plsc.load_gather(vinb, [iv0 + n]) * w1)
                    plsc.store_scatter(vm, [dr + c * OUT_H, dk], m)
                return carry

            lax.fori_loop(0, g1, p1_body, 0)

        def issue_out(t, vm, sem):
            _, r0, k0 = task_pos(t)
            pltpu.async_copy(vm, mh.at[pl.ds(r0, 448), pl.ds(k0, KP)], sem)

        def drain_out(vm, sem):
            pltpu.make_async_copy(mh.at[pl.ds(0, 448), pl.ds(0, KP)], vm,
                                  sem).wait()

        issue_in(0, vin0, sem_i0)

        def pair_body(t2, carry):
            for par, vinb, vm, sem_i, sem_i_nxt, vin_nxt, sem_o in (
                    (0, vin0, vma, sem_i0, sem_i1, vin1, sem_o0),
                    (1, vin1, vmb_, sem_i1, sem_i0, vin0, sem_o1)):
                t = t2 * 2 + par
                wait_in(t, vinb, sem_i)

                @pl.when(t + 1 < ntasks)
                def _():
                    issue_in(t + 1, vin_nxt, sem_i_nxt)

                @pl.when(t2 > 0)
                def _():
                    drain_out(vm, sem_o)

                compute(vinb, vm)
                issue_out(t, vm, sem_o)
            return carry

        lax.fori_loop(0, ntasks // 2, pair_body, 0)
        drain_out(vma, sem_o0)
        drain_out(vmb_, sem_o1)


def _tc_body(w1_ref, w2_ref, w3_ref, x0_ref, m1_ref, m2_ref, m3_ref, o_ref):
    cb = pl.program_id(1)

    @pl.when(cb < 3)
    def _():
        o_ref[...] = x0_ref[...]

    def expand(m_ref, w_ref):
        m = m_ref[...]
        for j in range(G):
            r = jax.lax.dot_general(m, w_ref[j], (((1,), (0,)), ((), ())),
                                    preferred_element_type=jnp.float32)
            o_ref[:, pl.ds(j * 8, 8)] = r.reshape(1, 8, OUT_H, OUT_W)

    @pl.when(jnp.logical_and(cb >= 3, cb < 9))
    def _():
        expand(m1_ref, w1_ref)

    @pl.when(jnp.logical_and(cb >= 9, cb < 21))
    def _():
        expand(m2_ref, w2_ref)

    @pl.when(cb >= 21)
    def _():
        expand(m3_ref, w3_ref)


@jax.jit
def _hypercolumns(x0, x1f, x2f, x3f, t1i, t1w, tdr, tdk, w1, w2, w3):
    mesh = plsc.VectorSubcoreMesh(core_axis_name="c", subcore_axis_name="s")
    mids = pl.kernel(
        _sc_body,
        out_type=(jax.ShapeDtypeStruct((MROWS[0], 128), jnp.float32),
                  jax.ShapeDtypeStruct((MROWS[1], 128), jnp.float32),
                  jax.ShapeDtypeStruct((MROWS[2], 128), jnp.float32)),
        mesh=mesh,
        compiler_params=pltpu.CompilerParams(use_tc_tiling_on_sc=False,
                                             needs_layout_passes=False),
        scratch_types=[
            pltpu.VMEM((P1_TOT,), jnp.int32),
            pltpu.VMEM((P1_TOT,), jnp.float32),
            pltpu.VMEM((P1_TOT,), jnp.int32),
            pltpu.VMEM((P1_TOT,), jnp.int32),
            pltpu.VMEM((CK * 28 * 28,), jnp.float32),
            pltpu.VMEM((CK * 28 * 28,), jnp.float32),
            pltpu.VMEM((CK * 14 * 14,), jnp.float32),
            pltpu.VMEM((CK * 14 * 14,), jnp.float32),
            pltpu.VMEM((CK * 7 * 7,), jnp.float32),
            pltpu.VMEM((CK * 7 * 7,), jnp.float32),
            pltpu.VMEM((448, KP), jnp.float32),
            pltpu.VMEM((448, KP), jnp.float32),
            pltpu.SemaphoreType.DMA,
            pltpu.SemaphoreType.DMA,
            pltpu.SemaphoreType.DMA,
            pltpu.SemaphoreType.DMA,
        ],
    )(x1f, x2f, x3f, t1i, t1w, tdr, tdk)
    m1, m2, m3 = mids
    out = pl.pallas_call(
        _tc_body,
        out_shape=jax.ShapeDtypeStruct((8, 1440, OUT_H, OUT_W), jnp.float32),
        grid=(8, NCB),
        in_specs=[
            pl.BlockSpec((G, 128, OUT_W), lambda b, cb: (0, 0, 0)),
            pl.BlockSpec((G, 128, OUT_W), lambda b, cb: (0, 0, 0)),
            pl.BlockSpec((G, 128, OUT_W), lambda b, cb: (0, 0, 0)),
            pl.BlockSpec((1, CBLK, OUT_H, OUT_W),
                         lambda b, cb: (b, jnp.minimum(cb, 2), 0, 0)),
            pl.BlockSpec((448, 128),
                         lambda b, cb: (b * 6 + jnp.clip(cb - 3, 0, 5), 0)),
            pl.BlockSpec((448, 128),
                         lambda b, cb: (b * 12 + jnp.clip(cb - 9, 0, 11), 0)),
            pl.BlockSpec((448, 128),
                         lambda b, cb: (b * 24 + jnp.clip(cb - 21, 0, 23), 0)),
        ],
        out_specs=pl.BlockSpec((1, CBLK, OUT_H, OUT_W),
                               lambda b, cb: (b, cb, 0, 0)),
    )(w1, w2, w3, x0, m1, m2, m3)
    return out


def kernel(x0, x1, x2, x3):
    t1i, t1w, tdr, tdk, (w1, w2, w3) = _tables_r5()
    return _hypercolumns(x0, x1.reshape(-1), x2.reshape(-1), x3.reshape(-1),
                         t1i, t1w, tdr, tdk, w1, w2, w3)
